# baseline (speedup 1.0000x reference)
import functools

import jax
import jax.numpy as jnp
from jax import lax
from jax.experimental import pallas as pl
from jax.experimental.pallas import tpu as pltpu
from jax.experimental.pallas import tpu_sc as plsc

E = 320000
D = 128
L = 16
NC = 2
NS = 16
NW = NC * NS
EPW = E // NW
C = 80
NCHUNK = EPW // C
GPC = C // L
KD = D // L


def _compute_chunk(rows_v, cols_v, outc_v, i):
    lane = lax.broadcasted_iota(jnp.int32, (L,), 0)

    SG = L // 4
    perm4 = (lane + 4) & (L - 1)

    def group_body(g, carry):
        sel = None
        for e in range(SG):
            eoff = g * SG + e
            acc = rows_v[eoff, pl.ds(0, L)] * cols_v[eoff, pl.ds(0, L)]
            for k in range(1, KD):
                acc = acc + (rows_v[eoff, pl.ds(k * L, L)]
                             * cols_v[eoff, pl.ds(k * L, L)])
            for shift in (8, 4, 2, 1):
                acc = acc + acc.at[lane ^ shift].get(mode="promise_in_bounds")
            sel = acc if e == 0 else jnp.where(lane == e, acc, sel)

        carry2 = jnp.where(
            lane < L - 4,
            carry.at[perm4].get(mode="promise_in_bounds"),
            sel.at[perm4].get(mode="promise_in_bounds"))

        @pl.when(g % 4 == 3)
        def _store():
            outc_v[pl.ds(i * C + (g // 4) * L, L)] = 1.0 / (1.0 + jnp.exp(-carry2))

        return carry2

    lax.fori_loop(0, 4 * GPC, group_body, jnp.zeros((L,), jnp.float32))


def _make_sc_call():
    mesh = plsc.VectorSubcoreMesh(core_axis_name="c", subcore_axis_name="s")

    @functools.partial(
        pl.kernel,
        mesh=mesh,
        out_type=jax.ShapeDtypeStruct((E,), jnp.float32),
        scratch_types=[
            pltpu.VMEM((NCHUNK, C), jnp.int32),
            pltpu.VMEM((NCHUNK, C), jnp.int32),
            pltpu.VMEM((C, D), jnp.float32),
            pltpu.VMEM((C, D), jnp.float32),
            pltpu.VMEM((C, D), jnp.float32),
            pltpu.VMEM((C, D), jnp.float32),
            pltpu.VMEM((EPW + 8,), jnp.float32),
            pltpu.SemaphoreType.DMA,
            pltpu.SemaphoreType.DMA,
            pltpu.SemaphoreType.DMA,
            pltpu.SemaphoreType.DMA,
        ],
    )
    def sc_call(z_hbm, row_hbm, col_hbm, out_hbm,
                ridx_v, cidx_v, rows_a, cols_a, rows_b, cols_b,
                outc_v, sem_ra, sem_ca, sem_rb, sem_cb):
        wid = lax.axis_index("s") * NC + lax.axis_index("c")
        pltpu.sync_copy(row_hbm.at[wid], ridx_v)
        pltpu.sync_copy(col_hbm.at[wid], cidx_v)

        def gather_start(i, rows, cols, sr, sc2):
            pltpu.make_async_copy(z_hbm.at[ridx_v.at[i]], rows, sr).start()
            pltpu.make_async_copy(z_hbm.at[cidx_v.at[i]], cols, sc2).start()

        def gather_wait(i, rows, cols, sr, sc2):
            pltpu.make_async_copy(z_hbm.at[ridx_v.at[i]], rows, sr).wait()
            pltpu.make_async_copy(z_hbm.at[cidx_v.at[i]], cols, sc2).wait()

        gather_start(0, rows_a, cols_a, sem_ra, sem_ca)

        def pair_body(j, carry):
            i0 = 2 * j
            i1 = i0 + 1
            gather_start(i1, rows_b, cols_b, sem_rb, sem_cb)
            gather_wait(i0, rows_a, cols_a, sem_ra, sem_ca)
            _compute_chunk(rows_a, cols_a, outc_v, i0)
            gather_start(i0 + 2, rows_a, cols_a, sem_ra, sem_ca)
            gather_wait(i1, rows_b, cols_b, sem_rb, sem_cb)
            _compute_chunk(rows_b, cols_b, outc_v, i1)
            return carry

        lax.fori_loop(0, (NCHUNK - 1) // 2, pair_body, 0)
        last = NCHUNK - 1
        gather_wait(last, rows_a, cols_a, sem_ra, sem_ca)
        _compute_chunk(rows_a, cols_a, outc_v, last)

        pltpu.sync_copy(outc_v.at[pl.ds(0, EPW)],
                        out_hbm.at[pl.ds(wid * EPW, EPW)])

    return sc_call


_SC_CALL = _make_sc_call()


def kernel(z, edge_index):
    ei = edge_index.astype(jnp.int32)
    row2 = ei[0].reshape(NW, NCHUNK, C)
    col2 = ei[1].reshape(NW, NCHUNK, C)
    return _SC_CALL(z, row2, col2)


# speedup vs baseline: 1.0392x; 1.0392x over previous
import functools

import jax
import jax.numpy as jnp
from jax import lax
from jax.experimental import pallas as pl
from jax.experimental.pallas import tpu as pltpu
from jax.experimental.pallas import tpu_sc as plsc

E = 320000
D = 128
L = 16
NC = 2
NS = 16
NW = NC * NS
EPW = E // NW
C = 80
NCHUNK = EPW // C
GPC = C // L
KD = D // L


def _compute_chunk(rows_v, cols_v, outc_v, i):
    lane = lax.broadcasted_iota(jnp.int32, (L,), 0)

    def group_body(g, sel_prev):
        sel = None
        for e in range(L // 2):
            eoff = g * (L // 2) + e
            acc = rows_v[eoff, pl.ds(0, L)] * cols_v[eoff, pl.ds(0, L)]
            for k in range(1, KD):
                acc = acc + (rows_v[eoff, pl.ds(k * L, L)]
                             * cols_v[eoff, pl.ds(k * L, L)])
            for shift in (8, 4, 2, 1):
                acc = acc + acc.at[lane ^ shift].get(mode="promise_in_bounds")
            sel = acc if e == 0 else jnp.where(lane == e, acc, sel)

        @pl.when(g % 2 == 1)
        def _store():
            hi = sel.at[lane ^ (L // 2)].get(mode="promise_in_bounds")
            both = jnp.where(lane < (L // 2), sel_prev, hi)
            outc_v[pl.ds(i * C + (g // 2) * L, L)] = 1.0 / (1.0 + jnp.exp(-both))

        return sel

    lax.fori_loop(0, 2 * GPC, group_body, jnp.zeros((L,), jnp.float32))


def _make_sc_call():
    mesh = plsc.VectorSubcoreMesh(core_axis_name="c", subcore_axis_name="s")

    @functools.partial(
        pl.kernel,
        mesh=mesh,
        out_type=jax.ShapeDtypeStruct((E,), jnp.float32),
        scratch_types=[
            pltpu.VMEM((NCHUNK, C), jnp.int32),
            pltpu.VMEM((NCHUNK, C), jnp.int32),
            pltpu.VMEM((C, D), jnp.float32),
            pltpu.VMEM((C, D), jnp.float32),
            pltpu.VMEM((C, D), jnp.float32),
            pltpu.VMEM((C, D), jnp.float32),
            pltpu.VMEM((EPW + 8,), jnp.float32),
            pltpu.SemaphoreType.DMA,
            pltpu.SemaphoreType.DMA,
            pltpu.SemaphoreType.DMA,
            pltpu.SemaphoreType.DMA,
        ],
    )
    def sc_call(z_hbm, row_hbm, col_hbm, out_hbm,
                ridx_v, cidx_v, rows_a, cols_a, rows_b, cols_b,
                outc_v, sem_ra, sem_ca, sem_rb, sem_cb):
        wid = lax.axis_index("s") * NC + lax.axis_index("c")
        pltpu.sync_copy(row_hbm.at[wid], ridx_v)
        pltpu.sync_copy(col_hbm.at[wid], cidx_v)

        def gather_start(i, rows, cols, sr, sc2):
            pltpu.make_async_copy(z_hbm.at[ridx_v.at[i]], rows, sr).start()
            pltpu.make_async_copy(z_hbm.at[cidx_v.at[i]], cols, sc2).start()

        def gather_wait(i, rows, cols, sr, sc2):
            pltpu.make_async_copy(z_hbm.at[ridx_v.at[i]], rows, sr).wait()
            pltpu.make_async_copy(z_hbm.at[cidx_v.at[i]], cols, sc2).wait()

        gather_start(0, rows_a, cols_a, sem_ra, sem_ca)

        def pair_body(j, carry):
            i0 = 2 * j
            i1 = i0 + 1
            gather_start(i1, rows_b, cols_b, sem_rb, sem_cb)
            gather_wait(i0, rows_a, cols_a, sem_ra, sem_ca)
            _compute_chunk(rows_a, cols_a, outc_v, i0)
            gather_start(i0 + 2, rows_a, cols_a, sem_ra, sem_ca)
            gather_wait(i1, rows_b, cols_b, sem_rb, sem_cb)
            _compute_chunk(rows_b, cols_b, outc_v, i1)
            return carry

        lax.fori_loop(0, (NCHUNK - 1) // 2, pair_body, 0)
        last = NCHUNK - 1
        gather_wait(last, rows_a, cols_a, sem_ra, sem_ca)
        _compute_chunk(rows_a, cols_a, outc_v, last)

        pltpu.sync_copy(outc_v.at[pl.ds(0, EPW)],
                        out_hbm.at[pl.ds(wid * EPW, EPW)])

    return sc_call


_SC_CALL = _make_sc_call()


def kernel(z, edge_index):
    ei = edge_index.astype(jnp.int32)
    row2 = ei[0].reshape(NW, NCHUNK, C)
    col2 = ei[1].reshape(NW, NCHUNK, C)
    return _SC_CALL(z, row2, col2)
